# baseline (speedup 1.0000x reference)
import functools

import jax
import jax.numpy as jnp
from jax import lax
from jax.experimental import pallas as pl
from jax.experimental.pallas import tpu as pltpu
from jax.experimental.pallas import tpu_sc as plsc

NUM_CORES = 2
NUM_SUBCORES = 16
LANES = 16
NW = NUM_CORES * NUM_SUBCORES

BATCH = 16384
N_FIELDS = 26
ROWS_PER_W = BATCH // NW
IDX_PER_W = ROWS_PER_W * N_FIELDS
IDX_ROWS = IDX_PER_W // LANES


TABLE = 1000000
STAGE_CHUNK = 62496
STAGE_TAIL = TABLE - 16 * STAGE_CHUNK
STAGE_ROUNDS = 4
STAGE_RB = STAGE_CHUNK // STAGE_ROUNDS


def _sc_lr(x3, w2d, bias1):
    mesh = plsc.VectorSubcoreMesh(core_axis_name="c", subcore_axis_name="s")

    @functools.partial(
        pl.kernel,
        out_type=jax.ShapeDtypeStruct((BATCH,), jnp.float32),
        mesh=mesh,
        compiler_params=pltpu.CompilerParams(needs_layout_passes=False),
        scratch_types=[
            pltpu.VMEM((IDX_PER_W,), jnp.int32),
            pltpu.VMEM((IDX_PER_W,), jnp.float32),
            pltpu.VMEM((LANES,), jnp.float32),
            pltpu.VMEM((ROWS_PER_W,), jnp.float32),
            pltpu.VMEM((STAGE_RB,), jnp.float32),
            pltpu.VMEM((STAGE_RB,), jnp.float32),
            pltpu.VMEM_SHARED((TABLE,), jnp.float32),
            pltpu.SemaphoreType.DMA,
            pltpu.SemaphoreType.DMA,
            pltpu.SemaphoreType.DMA,
        ],
    )
    def k(x_hbm, w_hbm, b_hbm, out_hbm, idx_v, val_v, bias_v, out_v,
          stage_a, stage_b, table_sp, sem, sem2, sem3):
        s = lax.axis_index("s")
        wid = s * NUM_CORES + lax.axis_index("c")
        wrow = w_hbm.at[0]
        off = s * STAGE_CHUNK
        bufs = (stage_a, stage_b)
        pltpu.async_copy(wrow.at[pl.ds(off, STAGE_RB)], bufs[0], sem2)
        pltpu.sync_copy(x_hbm.at[wid], idx_v)
        pltpu.sync_copy(b_hbm, bias_v)
        for r in range(STAGE_ROUNDS):
            ro = off + r * STAGE_RB
            pltpu.make_async_copy(wrow.at[pl.ds(ro, STAGE_RB)],
                                  bufs[r % 2], sem2).wait()
            if r >= 2:
                po = off + (r - 2) * STAGE_RB
                pltpu.make_async_copy(bufs[r % 2],
                                      table_sp.at[pl.ds(po, STAGE_RB)],
                                      sem3).wait()
            if r + 1 < STAGE_ROUNDS:
                pltpu.async_copy(wrow.at[pl.ds(ro + STAGE_RB, STAGE_RB)],
                                 bufs[(r + 1) % 2], sem2)
            pltpu.async_copy(bufs[r % 2], table_sp.at[pl.ds(ro, STAGE_RB)],
                             sem3)
        for r in (STAGE_ROUNDS - 2, STAGE_ROUNDS - 1):
            ro = off + r * STAGE_RB
            pltpu.make_async_copy(bufs[r % 2], table_sp.at[pl.ds(ro, STAGE_RB)],
                                  sem3).wait()

        @pl.when(s == 0)
        def _():
            tail0 = 16 * STAGE_CHUNK
            pltpu.sync_copy(wrow.at[pl.ds(tail0, STAGE_TAIL)],
                            stage_a.at[pl.ds(0, STAGE_TAIL)])
            pltpu.sync_copy(stage_a.at[pl.ds(0, STAGE_TAIL)],
                            table_sp.at[pl.ds(tail0, STAGE_TAIL)])

        plsc.subcore_barrier()
        half = IDX_PER_W // 2
        ga = pltpu.async_copy(table_sp.at[idx_v.at[pl.ds(0, half)]],
                              val_v.at[pl.ds(0, half)], sem)
        gb = pltpu.async_copy(table_sp.at[idx_v.at[pl.ds(half, half)]],
                              val_v.at[pl.ds(half, half)], sem2)

        bias_vec = bias_v[...]
        lane_f = lax.iota(jnp.int32, LANES) * N_FIELDS

        def jbody(j, _):
            base = j * (LANES * N_FIELDS)
            acc = bias_vec
            for f in range(N_FIELDS):
                acc = acc + plsc.load_gather(val_v, [lane_f + (base + f)])
            out_v[pl.ds(j * LANES, LANES)] = 1.0 / (1.0 + jnp.exp(-acc))
            return 0

        nj = ROWS_PER_W // LANES
        ga.wait()
        lax.fori_loop(0, nj // 2, jbody, 0)
        gb.wait()
        lax.fori_loop(nj // 2, nj, jbody, 0)
        pltpu.sync_copy(out_v, out_hbm.at[pl.ds(wid * ROWS_PER_W, ROWS_PER_W)])

    return k(x3, w2d, bias1)


def kernel(x, W, bias):
    x3 = x.reshape(NW, IDX_PER_W)
    w2d = W.reshape(1, TABLE)
    bias16 = jnp.broadcast_to(bias.astype(jnp.float32), (LANES,))
    out = _sc_lr(x3, w2d, bias16)
    return out.reshape(BATCH, 1)


# speedup vs baseline: 1.3606x; 1.3606x over previous
import functools

import jax
import jax.numpy as jnp
from jax import lax
from jax.experimental import pallas as pl
from jax.experimental.pallas import tpu as pltpu
from jax.experimental.pallas import tpu_sc as plsc

NUM_CORES = 2
NUM_SUBCORES = 16
LANES = 16
NW = NUM_CORES * NUM_SUBCORES

BATCH = 16384
N_FIELDS = 26
ROWS_PER_W = BATCH // NW
IDX_PER_W = ROWS_PER_W * N_FIELDS
IDX_ROWS = IDX_PER_W // LANES


TABLE = 1000000
STAGE_CHUNK = 62496
STAGE_TAIL = TABLE - 16 * STAGE_CHUNK
STAGE_ROUNDS = 4
STAGE_RB = STAGE_CHUNK // STAGE_ROUNDS


def _sc_lr(x3, w2d, bias1):
    mesh = plsc.VectorSubcoreMesh(core_axis_name="c", subcore_axis_name="s")

    @functools.partial(
        pl.kernel,
        out_type=jax.ShapeDtypeStruct((BATCH,), jnp.float32),
        mesh=mesh,
        compiler_params=pltpu.CompilerParams(needs_layout_passes=False),
        scratch_types=[
            pltpu.VMEM((IDX_PER_W,), jnp.int32),
            pltpu.VMEM((IDX_PER_W,), jnp.float32),
            pltpu.VMEM((LANES,), jnp.float32),
            pltpu.VMEM((ROWS_PER_W,), jnp.float32),
            pltpu.VMEM((STAGE_RB,), jnp.float32),
            pltpu.VMEM((STAGE_RB,), jnp.float32),
            pltpu.VMEM_SHARED((TABLE,), jnp.float32),
            pltpu.SemaphoreType.DMA,
            pltpu.SemaphoreType.DMA,
        ],
    )
    def k(x_hbm, w_hbm, b_hbm, out_hbm, idx_v, val_v, bias_v, out_v,
          stage_a, stage_b, table_sp, sem, sem2):
        s = lax.axis_index("s")
        wid = s * NUM_CORES + lax.axis_index("c")
        wrow = w_hbm.at[0]
        off = s * STAGE_CHUNK
        bufs = (stage_a, stage_b)
        pltpu.async_copy(wrow.at[pl.ds(off, STAGE_RB)], bufs[0], sem2)
        for f in range(N_FIELDS):
            pltpu.async_copy(
                x_hbm.at[pl.ds(f * BATCH + wid * ROWS_PER_W, ROWS_PER_W)],
                idx_v.at[pl.ds(f * ROWS_PER_W, ROWS_PER_W)], sem)
        pltpu.sync_copy(b_hbm, bias_v)
        for r in range(STAGE_ROUNDS):
            ro = off + r * STAGE_RB
            pltpu.make_async_copy(wrow.at[pl.ds(ro, STAGE_RB)],
                                  bufs[r % 2], sem2).wait()
            if r + 1 < STAGE_ROUNDS:
                pltpu.async_copy(wrow.at[pl.ds(ro + STAGE_RB, STAGE_RB)],
                                 bufs[(r + 1) % 2], sem2)
            pltpu.sync_copy(bufs[r % 2], table_sp.at[pl.ds(ro, STAGE_RB)])

        @pl.when(s == 0)
        def _():
            tail0 = 16 * STAGE_CHUNK
            pltpu.sync_copy(wrow.at[pl.ds(tail0, STAGE_TAIL)],
                            stage_a.at[pl.ds(0, STAGE_TAIL)])
            pltpu.sync_copy(stage_a.at[pl.ds(0, STAGE_TAIL)],
                            table_sp.at[pl.ds(tail0, STAGE_TAIL)])

        for f in range(N_FIELDS):
            pltpu.make_async_copy(
                x_hbm.at[pl.ds(f * BATCH + wid * ROWS_PER_W, ROWS_PER_W)],
                idx_v.at[pl.ds(f * ROWS_PER_W, ROWS_PER_W)], sem).wait()
        plsc.subcore_barrier()
        pltpu.async_copy(table_sp.at[idx_v], val_v, sem).wait()

        bias_vec = bias_v[...]

        def jbody(j, _):
            acc = bias_vec
            for f in range(N_FIELDS):
                acc = acc + val_v[pl.ds(f * ROWS_PER_W + j * LANES, LANES)]
            out_v[pl.ds(j * LANES, LANES)] = 1.0 / (1.0 + jnp.exp(-acc))
            return 0

        lax.fori_loop(0, ROWS_PER_W // LANES, jbody, 0)
        pltpu.sync_copy(out_v, out_hbm.at[pl.ds(wid * ROWS_PER_W, ROWS_PER_W)])

    return k(x3, w2d, bias1)


def kernel(x, W, bias):
    x3 = x.T.reshape(-1)
    w2d = W.reshape(1, TABLE)
    bias16 = jnp.broadcast_to(bias.astype(jnp.float32), (LANES,))
    out = _sc_lr(x3, w2d, bias16)
    return out.reshape(BATCH, 1)
